# baseline (speedup 1.0000x reference)
import functools

import jax
import jax.numpy as jnp
from jax import lax
from jax.experimental import pallas as pl
from jax.experimental.pallas import tpu as pltpu
from jax.experimental.pallas import tpu_sc as plsc

_N_ITEMS = 1000
_N_PERSONS = 100000
_N_GRADES = 5
_N_RESPONSES_REF = 2000000

_L = 16
_NC, _NS = 2, 16
_NW = _NC * _NS
_IT_PAD = 1008
_CHUNK = 4000
_GROUPS = _CHUNK // _L

_LN2 = 0.69314718055994530942


def _logf(x):
    xi = lax.bitcast_convert_type(x, jnp.int32)
    e = (xi >> 23) - 127
    m = lax.bitcast_convert_type(
        (xi & jnp.int32(0x007FFFFF)) | jnp.int32(0x3F800000), jnp.float32)
    z = m - 1.0
    p = jnp.float32(0.041550807862246186)
    for c in (-0.15783775201785516, 0.30656068045901524,
              -0.4970307397424109, 0.9999449867283122):
        p = p * z + jnp.float32(c)
    return z * p + e.astype(jnp.float32) * jnp.float32(_LN2)


def _softplus(x):
    return jnp.maximum(x, 0.0) + _logf(1.0 + jnp.exp(-jnp.abs(x)))


def _body(packed_hbm, a_hbm, bb_hbm, bd_hbm, t_hbm,
          out_hbm, t_v, a_v, b_v, bb_v, bd_v, pk0_v, pk1_v,
          part_v, sem0, sem1, *, n_chunks, pair_iters, prior_scale,
          with_prior):
    cid = lax.axis_index("c")
    sid = lax.axis_index("s")
    wid = sid * _NC + cid

    pltpu.sync_copy(t_hbm, t_v)
    pltpu.sync_copy(a_hbm, a_v)
    pltpu.sync_copy(bb_hbm, bb_v)
    pltpu.sync_copy(bd_hbm, bd_v)

    iota = lax.iota(jnp.int32, _L)
    widv = jnp.full((_L,), wid, jnp.int32)
    on_tile0 = widv == 0

    def tbody(i, pacc):
        rows = iota + i * _L
        a = _softplus(plsc.load_gather(a_v, [rows]))
        plsc.store_scatter(a_v, [rows], a)
        rows3 = rows * 3
        rows4 = rows * 4
        b0 = plsc.load_gather(bb_v, [rows])
        d0 = plsc.load_gather(bd_v, [rows3])
        d1 = plsc.load_gather(bd_v, [rows3 + 1])
        d2 = plsc.load_gather(bd_v, [rows3 + 2])
        b1 = b0 + _softplus(d0)
        b2 = b1 + _softplus(d1)
        b3 = b2 + _softplus(d2)
        plsc.store_scatter(b_v, [rows4], b0)
        plsc.store_scatter(b_v, [rows4 + 1], b1)
        plsc.store_scatter(b_v, [rows4 + 2], b2)
        plsc.store_scatter(b_v, [rows4 + 3], b3)
        if not with_prior:
            return pacc
        m0 = jnp.logical_and(rows < _N_ITEMS, on_tile0)
        sq = a * a + b0 * b0 + b1 * b1 + b2 * b2 + b3 * b3
        return pacc + jnp.where(m0, sq, 0.0)

    pacc = lax.fori_loop(0, _IT_PAD // _L, tbody,
                         jnp.zeros((_L,), jnp.float32))

    if with_prior:
        n_tg = _N_PERSONS // _L
        t_iters = -(-n_tg // _NW)

        def pbody(j, pacc):
            idxs = (wid + j * _NW) * _L + iota
            valid = idxs < _N_PERSONS
            tv = plsc.load_gather(t_v, [jnp.where(valid, idxs, 0)])
            return pacc + jnp.where(valid, tv * tv, 0.0)

        pacc = lax.fori_loop(0, t_iters, pbody, pacc)

    def start(c, buf):
        cc = jnp.minimum(c, n_chunks - 1)
        sem = sem0 if buf is pk0_v else sem1
        pltpu.async_copy(packed_hbm.at[pl.ds(cc * _CHUNK, _CHUNK)], buf, sem)

    def wait(buf):
        sem = sem0 if buf is pk0_v else sem1
        pltpu.make_async_copy(packed_hbm.at[pl.ds(0, _CHUNK)], buf, sem).wait()

    def process(buf, c):
        cvalid = jnp.full((_L,), c, jnp.int32) < n_chunks

        def gbody(g, carry):
            prod, esum = carry
            rows = iota + g * _L
            packed = plsc.load_gather(buf, [rows])
            item = packed & 0x3FF
            r = (packed >> 10) & 7
            person = packed >> 13
            tg = plsc.load_gather(t_v, [person])
            ag = plsc.load_gather(a_v, [item])
            item4 = item * 4
            bu = plsc.load_gather(b_v, [item4 + jnp.maximum(r - 1, 0)])
            bv = plsc.load_gather(b_v, [item4 + jnp.minimum(r, _N_GRADES - 2)])
            su = 1.0 / (1.0 + jnp.exp(ag * (bu - tg)))
            sv = 1.0 / (1.0 + jnp.exp(ag * (bv - tg)))
            plo = jnp.where(r == 0, 1.0, su)
            phi = jnp.where(r == _N_GRADES - 1, 0.0, sv)
            p = jnp.maximum(plo - phi, 1e-12)
            prod = prod * p
            bits = lax.bitcast_convert_type(prod, jnp.int32)
            esum = esum + (bits >> 23)
            prod = lax.bitcast_convert_type(
                (bits & jnp.int32(0x007FFFFF)) | jnp.int32(0x3F800000),
                jnp.float32)
            return prod, esum

        prod, esum = lax.fori_loop(
            0, _GROUPS, gbody,
            (jnp.ones((_L,), jnp.float32), jnp.zeros((_L,), jnp.int32)))
        chunk_acc = (_logf(prod)
                     + (esum - 127 * _GROUPS).astype(jnp.float32)
                     * jnp.float32(_LN2))
        return jnp.where(cvalid, chunk_acc, 0.0)

    iters = pair_iters * 2
    start(wid, pk0_v)

    def pair_body(j2, acc):
        j = j2 * 2
        c0 = wid + j * _NW
        start(c0 + _NW, pk1_v)
        wait(pk0_v)
        acc = acc + process(pk0_v, c0)

        @pl.when(j + 2 < iters)
        def _():
            start(c0 + 2 * _NW, pk0_v)

        wait(pk1_v)
        return acc + process(pk1_v, c0 + _NW)

    acc = lax.fori_loop(0, pair_iters, pair_body,
                        jnp.zeros((_L,), jnp.float32))

    part_v[...] = acc - jnp.float32(0.5 * prior_scale) * pacc
    pltpu.sync_copy(part_v, out_hbm.at[wid])


@functools.lru_cache(maxsize=None)
def _make_kernel(n_responses, total_responses, with_prior):
    n_chunks = n_responses // _CHUNK
    pair_iters = -(-n_chunks // (2 * _NW))
    prior_scale = total_responses / _N_RESPONSES_REF
    mesh = plsc.VectorSubcoreMesh(core_axis_name="c", subcore_axis_name="s")
    return pl.kernel(
        functools.partial(_body, n_chunks=n_chunks, pair_iters=pair_iters,
                          prior_scale=prior_scale, with_prior=with_prior),
        out_type=jax.ShapeDtypeStruct((_NW, _L), jnp.float32),
        mesh=mesh,
        compiler_params=pltpu.CompilerParams(
            needs_layout_passes=False, use_tc_tiling_on_sc=False),
        scratch_types=[
            pltpu.VMEM((_N_PERSONS,), jnp.float32),
            pltpu.VMEM((_IT_PAD,), jnp.float32),
            pltpu.VMEM((_IT_PAD * 4,), jnp.float32),
            pltpu.VMEM((_IT_PAD,), jnp.float32),
            pltpu.VMEM((_IT_PAD * 3,), jnp.float32),
            pltpu.VMEM((_CHUNK,), jnp.int32),
            pltpu.VMEM((_CHUNK,), jnp.int32),
            pltpu.VMEM((_L,), jnp.float32),
            pltpu.SemaphoreType.DMA,
            pltpu.SemaphoreType.DMA,
        ],
    )


_SPLITS = 4


def kernel(indices, a_, b_base_, b_diff_, t_):
    pad = _IT_PAD - a_.shape[0]
    a_p = jnp.pad(a_, (0, pad))
    bb_p = jnp.pad(b_base_.reshape(-1), (0, pad))
    bd_p = jnp.pad(b_diff_, ((0, pad), (0, 0))).reshape(-1)
    w = jnp.array([1, 1 << 13, 1 << 10], jnp.int32)
    n = indices.shape[0]
    rk = n // _SPLITS
    total = jnp.float32(0)
    for i in range(_SPLITS):
        sl = lax.slice_in_dim(indices, i * rk, (i + 1) * rk, axis=0)
        packed = lax.optimization_barrier(jnp.sum(sl * w[None, :], axis=1))
        parts = _make_kernel(rk, n, i == 0)(packed, a_p, bb_p, bd_p, t_)
        total = total + jnp.sum(parts)
    return -total


# speedup vs baseline: 1.1264x; 1.1264x over previous
import functools

import jax
import jax.numpy as jnp
from jax import lax
from jax.experimental import pallas as pl
from jax.experimental.pallas import tpu as pltpu
from jax.experimental.pallas import tpu_sc as plsc

_N_ITEMS = 1000
_N_PERSONS = 100000
_N_GRADES = 5
_N_RESPONSES_REF = 2000000

_L = 16
_NC, _NS = 2, 16
_NW = _NC * _NS
_IT_PAD = 1008
_CHUNK = 4000
_GROUPS = _CHUNK // _L

_LN2 = 0.69314718055994530942


def _logf(x):
    xi = lax.bitcast_convert_type(x, jnp.int32)
    e = (xi >> 23) - 127
    m = lax.bitcast_convert_type(
        (xi & jnp.int32(0x007FFFFF)) | jnp.int32(0x3F800000), jnp.float32)
    z = m - 1.0
    p = jnp.float32(0.041550807862246186)
    for c in (-0.15783775201785516, 0.30656068045901524,
              -0.4970307397424109, 0.9999449867283122):
        p = p * z + jnp.float32(c)
    return z * p + e.astype(jnp.float32) * jnp.float32(_LN2)


def _softplus(x):
    return jnp.maximum(x, 0.0) + _logf(1.0 + jnp.exp(-jnp.abs(x)))


def _body(packed_hbm, a_hbm, bb_hbm, bd_hbm, t_hbm,
          out_hbm, t_v, a_v, b_v, bb_v, bd_v, pk0_v, pk1_v,
          part_v, sem0, sem1, *, n_chunks, pair_iters, prior_scale,
          with_prior):
    cid = lax.axis_index("c")
    sid = lax.axis_index("s")
    wid = sid * _NC + cid

    pltpu.sync_copy(t_hbm, t_v)
    pltpu.sync_copy(a_hbm, a_v)
    pltpu.sync_copy(bb_hbm, bb_v)
    pltpu.sync_copy(bd_hbm, bd_v)

    iota = lax.iota(jnp.int32, _L)
    widv = jnp.full((_L,), wid, jnp.int32)
    on_tile0 = widv == 0

    def tbody(i, pacc):
        rows = iota + i * _L
        a = _softplus(plsc.load_gather(a_v, [rows]))
        plsc.store_scatter(a_v, [rows], a)
        rows3 = rows * 3
        rows4 = rows * 4
        b0 = plsc.load_gather(bb_v, [rows])
        d0 = plsc.load_gather(bd_v, [rows3])
        d1 = plsc.load_gather(bd_v, [rows3 + 1])
        d2 = plsc.load_gather(bd_v, [rows3 + 2])
        b1 = b0 + _softplus(d0)
        b2 = b1 + _softplus(d1)
        b3 = b2 + _softplus(d2)
        plsc.store_scatter(b_v, [rows4], b0)
        plsc.store_scatter(b_v, [rows4 + 1], b1)
        plsc.store_scatter(b_v, [rows4 + 2], b2)
        plsc.store_scatter(b_v, [rows4 + 3], b3)
        if not with_prior:
            return pacc
        m0 = jnp.logical_and(rows < _N_ITEMS, on_tile0)
        sq = a * a + b0 * b0 + b1 * b1 + b2 * b2 + b3 * b3
        return pacc + jnp.where(m0, sq, 0.0)

    pacc = lax.fori_loop(0, _IT_PAD // _L, tbody,
                         jnp.zeros((_L,), jnp.float32))

    if with_prior:
        n_tg = _N_PERSONS // _L
        t_iters = -(-n_tg // _NW)

        def pbody(j, pacc):
            idxs = (wid + j * _NW) * _L + iota
            valid = idxs < _N_PERSONS
            tv = plsc.load_gather(t_v, [jnp.where(valid, idxs, 0)])
            return pacc + jnp.where(valid, tv * tv, 0.0)

        pacc = lax.fori_loop(0, t_iters, pbody, pacc)

    def start(c, buf):
        cc = jnp.minimum(c, n_chunks - 1)
        sem = sem0 if buf is pk0_v else sem1
        pltpu.async_copy(packed_hbm.at[pl.ds(cc * _CHUNK, _CHUNK)], buf, sem)

    def wait(buf):
        sem = sem0 if buf is pk0_v else sem1
        pltpu.make_async_copy(packed_hbm.at[pl.ds(0, _CHUNK)], buf, sem).wait()

    def process(buf, c):
        cvalid = jnp.full((_L,), c, jnp.int32) < n_chunks

        def gbody(g, carry):
            prod, esum = carry
            rows = iota + g * _L
            packed = plsc.load_gather(buf, [rows])
            item = packed & 0x3FF
            r = (packed >> 10) & 7
            person = packed >> 13
            tg = plsc.load_gather(t_v, [person])
            ag = plsc.load_gather(a_v, [item])
            item4 = item * 4
            bu = plsc.load_gather(b_v, [item4 + jnp.maximum(r - 1, 0)])
            bv = plsc.load_gather(b_v, [item4 + jnp.minimum(r, _N_GRADES - 2)])
            su = 1.0 / (1.0 + jnp.exp(ag * (bu - tg)))
            sv = 1.0 / (1.0 + jnp.exp(ag * (bv - tg)))
            plo = jnp.where(r == 0, 1.0, su)
            phi = jnp.where(r == _N_GRADES - 1, 0.0, sv)
            p = jnp.maximum(plo - phi, 1e-12)
            prod = prod * p
            bits = lax.bitcast_convert_type(prod, jnp.int32)
            esum = esum + (bits >> 23)
            prod = lax.bitcast_convert_type(
                (bits & jnp.int32(0x007FFFFF)) | jnp.int32(0x3F800000),
                jnp.float32)
            return prod, esum

        prod, esum = lax.fori_loop(
            0, _GROUPS, gbody,
            (jnp.ones((_L,), jnp.float32), jnp.zeros((_L,), jnp.int32)))
        chunk_acc = (_logf(prod)
                     + (esum - 127 * _GROUPS).astype(jnp.float32)
                     * jnp.float32(_LN2))
        return jnp.where(cvalid, chunk_acc, 0.0)

    iters = pair_iters * 2
    start(wid, pk0_v)

    def pair_body(j2, acc):
        j = j2 * 2
        c0 = wid + j * _NW
        start(c0 + _NW, pk1_v)
        wait(pk0_v)
        acc = acc + process(pk0_v, c0)

        @pl.when(j + 2 < iters)
        def _():
            start(c0 + 2 * _NW, pk0_v)

        wait(pk1_v)
        return acc + process(pk1_v, c0 + _NW)

    acc = lax.fori_loop(0, pair_iters, pair_body,
                        jnp.zeros((_L,), jnp.float32))

    part_v[...] = acc - jnp.float32(0.5 * prior_scale) * pacc
    pltpu.sync_copy(part_v, out_hbm.at[wid])


@functools.lru_cache(maxsize=None)
def _make_kernel(n_responses, total_responses, with_prior):
    n_chunks = n_responses // _CHUNK
    pair_iters = -(-n_chunks // (2 * _NW))
    prior_scale = total_responses / _N_RESPONSES_REF
    mesh = plsc.VectorSubcoreMesh(core_axis_name="c", subcore_axis_name="s")
    return pl.kernel(
        functools.partial(_body, n_chunks=n_chunks, pair_iters=pair_iters,
                          prior_scale=prior_scale, with_prior=with_prior),
        out_type=jax.ShapeDtypeStruct((_NW, _L), jnp.float32),
        mesh=mesh,
        compiler_params=pltpu.CompilerParams(
            needs_layout_passes=False, use_tc_tiling_on_sc=False),
        scratch_types=[
            pltpu.VMEM((_N_PERSONS,), jnp.float32),
            pltpu.VMEM((_IT_PAD,), jnp.float32),
            pltpu.VMEM((_IT_PAD * 4,), jnp.float32),
            pltpu.VMEM((_IT_PAD,), jnp.float32),
            pltpu.VMEM((_IT_PAD * 3,), jnp.float32),
            pltpu.VMEM((_CHUNK,), jnp.int32),
            pltpu.VMEM((_CHUNK,), jnp.int32),
            pltpu.VMEM((_L,), jnp.float32),
            pltpu.SemaphoreType.DMA,
            pltpu.SemaphoreType.DMA,
        ],
    )


_SPLIT_FRACS = (0.372, 0.326, 0.302)


def kernel(indices, a_, b_base_, b_diff_, t_):
    pad = _IT_PAD - a_.shape[0]
    a_p = jnp.pad(a_, (0, pad))
    bb_p = jnp.pad(b_base_.reshape(-1), (0, pad))
    bd_p = jnp.pad(b_diff_, ((0, pad), (0, 0))).reshape(-1)
    w = jnp.array([1, 1 << 13, 1 << 10], jnp.int32)
    n = indices.shape[0]
    sizes = [int(f * n) // _CHUNK * _CHUNK for f in _SPLIT_FRACS[:-1]]
    sizes.append(n - sum(sizes))
    total = jnp.float32(0)
    off = 0
    for i, sz in enumerate(sizes):
        sl = lax.slice_in_dim(indices, off, off + sz, axis=0)
        off += sz
        packed = lax.optimization_barrier(jnp.sum(sl * w[None, :], axis=1))
        parts = _make_kernel(sz, n, i == 0)(packed, a_p, bb_p, bd_p, t_)
        total = total + jnp.sum(parts)
    return -total


# speedup vs baseline: 1.1907x; 1.0571x over previous
import functools

import jax
import jax.numpy as jnp
from jax import lax
from jax.experimental import pallas as pl
from jax.experimental.pallas import tpu as pltpu
from jax.experimental.pallas import tpu_sc as plsc

_N_ITEMS = 1000
_N_PERSONS = 100000
_N_GRADES = 5
_N_RESPONSES_REF = 2000000

_L = 16
_NC, _NS = 2, 16
_NW = _NC * _NS
_IT_PAD = 1008
_CHUNK = 4000
_GROUPS = _CHUNK // _L

_LN2 = 0.69314718055994530942


def _logf(x):
    xi = lax.bitcast_convert_type(x, jnp.int32)
    e = (xi >> 23) - 127
    m = lax.bitcast_convert_type(
        (xi & jnp.int32(0x007FFFFF)) | jnp.int32(0x3F800000), jnp.float32)
    z = m - 1.0
    p = jnp.float32(0.041550807862246186)
    for c in (-0.15783775201785516, 0.30656068045901524,
              -0.4970307397424109, 0.9999449867283122):
        p = p * z + jnp.float32(c)
    return z * p + e.astype(jnp.float32) * jnp.float32(_LN2)


def _softplus(x):
    return jnp.maximum(x, 0.0) + _logf(1.0 + jnp.exp(-jnp.abs(x)))


def _body(packed_hbm, a_hbm, bb_hbm, bd_hbm, t_hbm,
          out_hbm, t_v, a_v, b_v, bb_v, bd_v, pk0_v, pk1_v,
          part_v, sem0, sem1, *, n_chunks, pair_iters, prior_scale,
          with_prior):
    cid = lax.axis_index("c")
    sid = lax.axis_index("s")
    wid = sid * _NC + cid

    pltpu.sync_copy(t_hbm, t_v)
    pltpu.sync_copy(a_hbm, a_v)
    pltpu.sync_copy(bb_hbm, bb_v)
    pltpu.sync_copy(bd_hbm, bd_v)

    iota = lax.iota(jnp.int32, _L)
    widv = jnp.full((_L,), wid, jnp.int32)
    on_tile0 = widv == 0

    def tbody(i, pacc):
        rows = iota + i * _L
        a = _softplus(plsc.load_gather(a_v, [rows]))
        plsc.store_scatter(a_v, [rows], a)
        rows3 = rows * 3
        rows4 = rows * 4
        b0 = plsc.load_gather(bb_v, [rows])
        d0 = plsc.load_gather(bd_v, [rows3])
        d1 = plsc.load_gather(bd_v, [rows3 + 1])
        d2 = plsc.load_gather(bd_v, [rows3 + 2])
        b1 = b0 + _softplus(d0)
        b2 = b1 + _softplus(d1)
        b3 = b2 + _softplus(d2)
        plsc.store_scatter(b_v, [rows4], b0)
        plsc.store_scatter(b_v, [rows4 + 1], b1)
        plsc.store_scatter(b_v, [rows4 + 2], b2)
        plsc.store_scatter(b_v, [rows4 + 3], b3)
        if not with_prior:
            return pacc
        m0 = jnp.logical_and(rows < _N_ITEMS, on_tile0)
        sq = a * a + b0 * b0 + b1 * b1 + b2 * b2 + b3 * b3
        return pacc + jnp.where(m0, sq, 0.0)

    pacc = lax.fori_loop(0, _IT_PAD // _L, tbody,
                         jnp.zeros((_L,), jnp.float32))

    if with_prior:
        n_tg = _N_PERSONS // _L
        t_iters = -(-n_tg // _NW)

        def pbody(j, pacc):
            idxs = (wid + j * _NW) * _L + iota
            valid = idxs < _N_PERSONS
            tv = plsc.load_gather(t_v, [jnp.where(valid, idxs, 0)])
            return pacc + jnp.where(valid, tv * tv, 0.0)

        pacc = lax.fori_loop(0, t_iters, pbody, pacc)

    def start(c, buf):
        cc = jnp.minimum(c, n_chunks - 1)
        sem = sem0 if buf is pk0_v else sem1
        pltpu.async_copy(packed_hbm.at[pl.ds(cc * _CHUNK, _CHUNK)], buf, sem)

    def wait(buf):
        sem = sem0 if buf is pk0_v else sem1
        pltpu.make_async_copy(packed_hbm.at[pl.ds(0, _CHUNK)], buf, sem).wait()

    def process(buf, c):
        cvalid = jnp.full((_L,), c, jnp.int32) < n_chunks

        def gbody(g, carry):
            prod, esum = carry
            rows = iota + g * _L
            packed = plsc.load_gather(buf, [rows])
            item = packed & 0x3FF
            r = (packed >> 10) & 7
            person = packed >> 13
            tg = plsc.load_gather(t_v, [person])
            ag = plsc.load_gather(a_v, [item])
            item4 = item * 4
            bu = plsc.load_gather(b_v, [item4 + jnp.maximum(r - 1, 0)])
            bv = plsc.load_gather(b_v, [item4 + jnp.minimum(r, _N_GRADES - 2)])
            su = 1.0 / (1.0 + jnp.exp(ag * (bu - tg)))
            sv = 1.0 / (1.0 + jnp.exp(ag * (bv - tg)))
            plo = jnp.where(r == 0, 1.0, su)
            phi = jnp.where(r == _N_GRADES - 1, 0.0, sv)
            p = jnp.maximum(plo - phi, 1e-12)
            prod = prod * p
            bits = lax.bitcast_convert_type(prod, jnp.int32)
            esum = esum + (bits >> 23)
            prod = lax.bitcast_convert_type(
                (bits & jnp.int32(0x007FFFFF)) | jnp.int32(0x3F800000),
                jnp.float32)
            return prod, esum

        prod, esum = lax.fori_loop(
            0, _GROUPS, gbody,
            (jnp.ones((_L,), jnp.float32), jnp.zeros((_L,), jnp.int32)))
        chunk_acc = (_logf(prod)
                     + (esum - 127 * _GROUPS).astype(jnp.float32)
                     * jnp.float32(_LN2))
        return jnp.where(cvalid, chunk_acc, 0.0)

    iters = pair_iters * 2
    start(wid, pk0_v)

    def pair_body(j2, acc):
        j = j2 * 2
        c0 = wid + j * _NW
        start(c0 + _NW, pk1_v)
        wait(pk0_v)
        acc = acc + process(pk0_v, c0)

        @pl.when(j + 2 < iters)
        def _():
            start(c0 + 2 * _NW, pk0_v)

        wait(pk1_v)
        return acc + process(pk1_v, c0 + _NW)

    acc = lax.fori_loop(0, pair_iters, pair_body,
                        jnp.zeros((_L,), jnp.float32))

    part_v[...] = acc - jnp.float32(0.5 * prior_scale) * pacc
    pltpu.sync_copy(part_v, out_hbm.at[wid])


@functools.lru_cache(maxsize=None)
def _make_kernel(n_responses, total_responses, with_prior):
    n_chunks = n_responses // _CHUNK
    pair_iters = -(-n_chunks // (2 * _NW))
    prior_scale = total_responses / _N_RESPONSES_REF
    mesh = plsc.VectorSubcoreMesh(core_axis_name="c", subcore_axis_name="s")
    return pl.kernel(
        functools.partial(_body, n_chunks=n_chunks, pair_iters=pair_iters,
                          prior_scale=prior_scale, with_prior=with_prior),
        out_type=jax.ShapeDtypeStruct((_NW, _L), jnp.float32),
        mesh=mesh,
        compiler_params=pltpu.CompilerParams(
            needs_layout_passes=False, use_tc_tiling_on_sc=False),
        scratch_types=[
            pltpu.VMEM((_N_PERSONS,), jnp.float32),
            pltpu.VMEM((_IT_PAD,), jnp.float32),
            pltpu.VMEM((_IT_PAD * 4,), jnp.float32),
            pltpu.VMEM((_IT_PAD,), jnp.float32),
            pltpu.VMEM((_IT_PAD * 3,), jnp.float32),
            pltpu.VMEM((_CHUNK,), jnp.int32),
            pltpu.VMEM((_CHUNK,), jnp.int32),
            pltpu.VMEM((_L,), jnp.float32),
            pltpu.SemaphoreType.DMA,
            pltpu.SemaphoreType.DMA,
        ],
    )


_SPLIT_FRACS = (0.57, 0.43)


def kernel(indices, a_, b_base_, b_diff_, t_):
    pad = _IT_PAD - a_.shape[0]
    a_p = jnp.pad(a_, (0, pad))
    bb_p = jnp.pad(b_base_.reshape(-1), (0, pad))
    bd_p = jnp.pad(b_diff_, ((0, pad), (0, 0))).reshape(-1)
    w = jnp.array([1, 1 << 13, 1 << 10], jnp.int32)
    n = indices.shape[0]
    sizes = [int(f * n) // _CHUNK * _CHUNK for f in _SPLIT_FRACS[:-1]]
    sizes.append(n - sum(sizes))
    total = jnp.float32(0)
    off = 0
    for i, sz in enumerate(sizes):
        sl = lax.slice_in_dim(indices, off, off + sz, axis=0)
        off += sz
        packed = lax.optimization_barrier(jnp.sum(sl * w[None, :], axis=1))
        parts = _make_kernel(sz, n, i == 0)(packed, a_p, bb_p, bd_p, t_)
        total = total + jnp.sum(parts)
    return -total
